# baseline (speedup 1.0000x reference)
import jax
import jax.numpy as jnp
from jax import lax
from jax.experimental import pallas as pl
from jax.experimental.pallas import tpu as pltpu
from jax.experimental.pallas import tpu_sc as plsc

N_PTS = 1048576
NUM_BATCHES = 8
RES = 128
EMA_DECAY = 0.95

NT = 32
NOWN = 64
CHUNK = N_PTS // NT
QTR = CHUNK // 4
NREG = N_PTS // QTR
WIN = 2048
C = 512
OROW = NOWN * 16 + 16


def _wid():
    return lax.axis_index("s") * 2 + lax.axis_index("c")


def _mesh():
    return plsc.VectorSubcoreMesh(core_axis_name="c", subcore_axis_name="s")


def _writer_body(px_hbm, py_hbm, pz_hbm, bidx_hbm, occ_hbm,
                 skey_hbm, sval_hbm, offs_hbm,
                 pxa, pya, pza, bxa, pxb, pyb, pzb, bxb,
                 keys, vals, hist, curs, offsb,
                 skey, sval, sem_a, sem_b, sem_v):
    wid = _wid()
    lane = lax.iota(jnp.int32, 16)
    c0 = jnp.zeros((16,), jnp.float32)
    ones = jnp.ones((16,), jnp.float32)
    NW = QTR // WIN

    def stage(q, w, px, py, pz, bx, sem):
        p0 = wid * CHUNK + q * QTR + w * WIN
        pltpu.async_copy(px_hbm.at[pl.ds(p0, WIN)], px, sem)
        pltpu.async_copy(py_hbm.at[pl.ds(p0, WIN)], py, sem)
        pltpu.async_copy(pz_hbm.at[pl.ds(p0, WIN)], pz, sem)
        pltpu.async_copy(bidx_hbm.at[pl.ds(p0, WIN)], bx, sem)
        pltpu.async_copy(occ_hbm.at[pl.ds(p0, WIN)],
                         vals.at[pl.ds(w * WIN, WIN)], sem_v)

    def wait4(px, py, pz, bx, sem):
        pltpu.make_async_copy(px_hbm.at[pl.ds(0, WIN)], px, sem).wait()
        pltpu.make_async_copy(px_hbm.at[pl.ds(0, WIN)], py, sem).wait()
        pltpu.make_async_copy(px_hbm.at[pl.ds(0, WIN)], pz, sem).wait()
        pltpu.make_async_copy(bidx_hbm.at[pl.ds(0, WIN)], bx, sem).wait()

    def quarter(q, _):
        def zh(j, _):
            hist[pl.ds(j * 16, 16)] = c0
            return 0
        lax.fori_loop(0, NOWN, zh, 0)

        def compute(w, px, py, pz, bx):
            def vreg(i, _):
                x = px[pl.ds(i * 16, 16)]
                y = py[pl.ds(i * 16, 16)]
                z = pz[pl.ds(i * 16, 16)]
                b = bx[pl.ds(i * 16, 16)]
                vx = jnp.clip((x * 64.0 + 64.0).astype(jnp.int32), 64, 127) - 64
                vy = jnp.clip((y * 64.0 + 64.0).astype(jnp.int32), 64, 127) - 64
                vz = jnp.clip((z * 64.0 + 64.0).astype(jnp.int32), 64, 127) - 64
                key = ((b * 64 + vx) * 64 + vy) * 64 + vz
                keys[pl.ds(w * WIN + i * 16, 16)] = key
                hidx = lax.shift_right_logical(key, 15) * 16 + lane
                plsc.addupdate_scatter(hist, [hidx], ones)
                return 0
            lax.fori_loop(0, WIN // 16, vreg, 0)

        stage(q, 0, pxa, pya, pza, bxa, sem_a)

        def winpair(wi, _):
            w = 2 * wi
            stage(q, w + 1, pxb, pyb, pzb, bxb, sem_b)
            wait4(pxa, pya, pza, bxa, sem_a)
            compute(w, pxa, pya, pza, bxa)

            @pl.when(w + 2 < NW)
            def _():
                stage(q, w + 2, pxa, pya, pza, bxa, sem_a)
            wait4(pxb, pyb, pzb, bxb, sem_b)
            compute(w + 1, pxb, pyb, pzb, bxb)
            return 0
        lax.fori_loop(0, NW // 2, winpair, 0)

        def drainv(w, _):
            pltpu.make_async_copy(occ_hbm.at[pl.ds(0, WIN)],
                                  vals.at[pl.ds(w * WIN, WIN)], sem_v).wait()
            return 0
        lax.fori_loop(0, NW, drainv, 0)

        def pfx(j, run):
            hv = hist[pl.ds(j * 16, 16)].astype(jnp.int32)
            inc = plsc.cumsum(hv)
            ex = inc - hv + run
            curs[pl.ds(j * 16, 16)] = ex
            offsb[pl.ds(j * 16, 16)] = ex
            return run + jnp.sum(hv)
        lax.fori_loop(0, NOWN, pfx, jnp.int32(0))

        def perm(i, _):
            k = keys[pl.ds(i * 16, 16)]
            v = vals[pl.ds(i * 16, 16)]
            hidx = lax.shift_right_logical(k, 15) * 16 + lane
            cur = plsc.load_gather(curs, [hidx])
            plsc.store_scatter(skey, [cur], k)
            plsc.store_scatter(sval, [cur], v)
            plsc.store_scatter(curs, [hidx], cur + 1)
            return 0
        lax.fori_loop(0, QTR // 16, perm, 0)

        r = wid * 4 + q
        pltpu.sync_copy(skey, skey_hbm.at[pl.ds(r * QTR, QTR)])
        pltpu.sync_copy(sval, sval_hbm.at[pl.ds(r * QTR, QTR)])
        pltpu.sync_copy(offsb, offs_hbm.at[pl.ds(r * OROW, OROW)])
        return 0

    lax.fori_loop(0, 4, quarter, 0)


def _writer(px, py, pz, bidx, occ_val):
    f = pl.kernel(
        _writer_body,
        out_type=(
            jax.ShapeDtypeStruct((N_PTS,), jnp.int32),
            jax.ShapeDtypeStruct((N_PTS,), jnp.float32),
            jax.ShapeDtypeStruct((NREG * OROW,), jnp.int32),
        ),
        mesh=_mesh(),
        compiler_params=pltpu.CompilerParams(needs_layout_passes=False),
        scratch_types=[
            pltpu.VMEM((WIN,), jnp.float32),
            pltpu.VMEM((WIN,), jnp.float32),
            pltpu.VMEM((WIN,), jnp.float32),
            pltpu.VMEM((WIN,), jnp.int32),
            pltpu.VMEM((WIN,), jnp.float32),
            pltpu.VMEM((WIN,), jnp.float32),
            pltpu.VMEM((WIN,), jnp.float32),
            pltpu.VMEM((WIN,), jnp.int32),
            pltpu.VMEM((QTR,), jnp.int32),
            pltpu.VMEM((QTR,), jnp.float32),
            pltpu.VMEM((NOWN * 16,), jnp.float32),
            pltpu.VMEM((NOWN * 16,), jnp.int32),
            pltpu.VMEM((OROW,), jnp.int32),
            pltpu.VMEM((QTR,), jnp.int32),
            pltpu.VMEM((QTR,), jnp.float32),
            pltpu.SemaphoreType.DMA,
            pltpu.SemaphoreType.DMA,
            pltpu.SemaphoreType.DMA,
        ],
    )
    return f(px, py, pz, bidx, occ_val)


GRP = 16


def _reader_body(neg_hbm, skey_hbm, sval_hbm, offs_hbm, sgrid_hbm,
                 s3, kbuf_a, vbuf_a, kbuf_b, vbuf_b,
                 kbuf_c, vbuf_c, kbuf_d, vbuf_d, offsg,
                 mystart, myend, sem_a, sem_b, sem_c, sem_d):
    wid = _wid()
    lane = lax.iota(jnp.int32, 16)
    zl = jnp.zeros((16,), jnp.int32)

    def owner(sub, _):
        o = wid * 2 + sub
        is_last = o == NOWN - 1

        pltpu.sync_copy(neg_hbm, s3)

        def group(g, _):
            pltpu.sync_copy(offs_hbm.at[pl.ds(g * GRP * OROW, GRP * OROW)],
                            offsg)
            gidx = lane * OROW + o * 16
            sv = plsc.load_gather(offsg, [gidx])
            ev = jnp.where(is_last, QTR, plsc.load_gather(offsg, [gidx + 16]))
            mystart[pl.ds(g * 16, 16)] = sv
            myend[pl.ds(g * 16, 16)] = ev
            return 0
        lax.fori_loop(0, NREG // GRP, group, 0)

        def bounds(r):
            ridx = zl + r
            start = plsc.load_gather(mystart, [ridx])[0]
            end = plsc.load_gather(myend, [ridx])[0]
            start8 = jnp.bitwise_and(start, ~7)
            nominal = r * QTR + start8
            goff = pl.multiple_of(jnp.minimum(nominal, N_PTS - C), 8)
            return start8, end, nominal, goff

        def issue(r, kb, vb, sem):
            _, _, _, goff = bounds(r)
            pltpu.async_copy(skey_hbm.at[pl.ds(goff, C)], kb, sem)
            pltpu.async_copy(sval_hbm.at[pl.ds(goff, C)], vb, sem)

        def do_vregs(kb, vb, va, vb_hi):
            def vreg(k, _):
                kk = kb[pl.ds(k * 16, 16)]
                vv = vb[pl.ds(k * 16, 16)]
                m = lax.shift_right_logical(kk, 15) == o
                rel = jnp.bitwise_and(kk, 32767)
                i0 = lax.shift_right_logical(rel, 12)
                i1 = jnp.bitwise_and(lax.shift_right_logical(rel, 6), 63)
                i2 = jnp.bitwise_and(rel, 63)

                def cond(mm):
                    return jnp.any(mm)

                def body(mm):
                    cur = plsc.load_gather(s3, [i0, i1, i2], mask=mm)
                    upd = jnp.logical_and(mm, vv > cur)
                    plsc.store_scatter(s3, [i0, i1, i2], vv, mask=upd)
                    cur2 = plsc.load_gather(s3, [i0, i1, i2], mask=mm)
                    return jnp.logical_and(mm, vv > cur2)
                lax.while_loop(cond, body, m)
                return 0
            lax.fori_loop(va, vb_hi, vreg, 0)

        def process(r, kb, vb, sem):
            start8, end, nominal, goff = bounds(r)
            pltpu.make_async_copy(skey_hbm.at[pl.ds(0, C)], kb, sem).wait()
            pltpu.make_async_copy(sval_hbm.at[pl.ds(0, C)], vb, sem).wait()
            a = nominal - goff
            p1 = jnp.minimum(end, start8 + C)
            bb = r * QTR + p1 - goff
            do_vregs(kb, vb, lax.shift_right_logical(a, 4),
                     lax.shift_right_logical(bb + 15, 4))
            nch = (end - start8 + C - 1) // C

            def chunk(j, _):
                nom_j = r * QTR + start8 + j * C
                goff_j = pl.multiple_of(jnp.minimum(nom_j, N_PTS - C), 8)
                pltpu.sync_copy(skey_hbm.at[pl.ds(goff_j, C)], kb)
                pltpu.sync_copy(sval_hbm.at[pl.ds(goff_j, C)], vb)
                aj = nom_j - goff_j
                pj = jnp.minimum(end, start8 + (j + 1) * C)
                bj = r * QTR + pj - goff_j
                do_vregs(kb, vb, lax.shift_right_logical(aj, 4),
                         lax.shift_right_logical(bj + 15, 4))
                return 0
            lax.fori_loop(1, nch, chunk, 0)

        roff = wid * 4 + sub * 2

        def rmap(r):
            return jnp.bitwise_and(r + roff, NREG - 1)

        issue(rmap(0), kbuf_a, vbuf_a, sem_a)
        issue(rmap(1), kbuf_b, vbuf_b, sem_b)
        issue(rmap(2), kbuf_c, vbuf_c, sem_c)

        def quad(t, _):
            r = 4 * t
            issue(rmap(r + 3), kbuf_d, vbuf_d, sem_d)
            process(rmap(r), kbuf_a, vbuf_a, sem_a)

            @pl.when(t < NREG // 4 - 1)
            def _():
                issue(rmap(r + 4), kbuf_a, vbuf_a, sem_a)
            process(rmap(r + 1), kbuf_b, vbuf_b, sem_b)

            @pl.when(t < NREG // 4 - 1)
            def _():
                issue(rmap(r + 5), kbuf_b, vbuf_b, sem_b)
            process(rmap(r + 2), kbuf_c, vbuf_c, sem_c)

            @pl.when(t < NREG // 4 - 1)
            def _():
                issue(rmap(r + 6), kbuf_c, vbuf_c, sem_c)
            process(rmap(r + 3), kbuf_d, vbuf_d, sem_d)
            return 0
        lax.fori_loop(0, NREG // 4, quad, 0)

        b = lax.shift_right_logical(o, 3)
        xi0 = jnp.bitwise_and(o, 7) * 8
        pltpu.sync_copy(s3, sgrid_hbm.at[b, pl.ds(xi0, 8)])
        return 0

    lax.fori_loop(0, 2, owner, 0)


def _reader(neg, skey, sval, offs):
    f = pl.kernel(
        _reader_body,
        out_type=jax.ShapeDtypeStruct((NUM_BATCHES, 64, 64, 64), jnp.float32),
        mesh=_mesh(),
        compiler_params=pltpu.CompilerParams(needs_layout_passes=False),
        scratch_types=[
            pltpu.VMEM((8, 64, 64), jnp.float32),
            pltpu.VMEM((C,), jnp.int32),
            pltpu.VMEM((C,), jnp.float32),
            pltpu.VMEM((C,), jnp.int32),
            pltpu.VMEM((C,), jnp.float32),
            pltpu.VMEM((C,), jnp.int32),
            pltpu.VMEM((C,), jnp.float32),
            pltpu.VMEM((C,), jnp.int32),
            pltpu.VMEM((C,), jnp.float32),
            pltpu.VMEM((GRP * OROW,), jnp.int32),
            pltpu.VMEM((NREG,), jnp.int32),
            pltpu.VMEM((NREG,), jnp.int32),
            pltpu.SemaphoreType.DMA,
            pltpu.SemaphoreType.DMA,
            pltpu.SemaphoreType.DMA,
            pltpu.SemaphoreType.DMA,
        ],
    )
    return f(neg, skey, sval, offs)


XBLK = 16


def _copy_body(g_ref, out_ref):
    out_ref[...] = g_ref[...]


def _copy(grid):
    return pl.pallas_call(
        _copy_body,
        grid=(NUM_BATCHES, RES // XBLK),
        in_specs=[pl.BlockSpec((1, XBLK, RES, RES), lambda b, x: (b, x, 0, 0))],
        out_specs=pl.BlockSpec((1, XBLK, RES, RES), lambda b, x: (b, x, 0, 0)),
        out_shape=jax.ShapeDtypeStruct(
            (NUM_BATCHES, RES, RES, RES), jnp.float32),
    )(grid)


def _oct_body(base_ref, s_ref, out_ref):
    gq = base_ref[0, :, :, 64:128]
    s = s_ref[0]
    out_ref[0, :, :, 0:64] = base_ref[0, :, :, 0:64]
    out_ref[0, :, :, 64:128] = jnp.where(
        s >= 0.0, jnp.maximum(jnp.float32(EMA_DECAY) * gq, s), gq)


def _octant(base, sgrid):
    return pl.pallas_call(
        _oct_body,
        grid=(NUM_BATCHES, 64 // XBLK),
        in_specs=[
            pl.BlockSpec((1, XBLK, 64, RES),
                         lambda b, x: (b, x + 64 // XBLK, 1, 0)),
            pl.BlockSpec((1, XBLK, 64, 64), lambda b, x: (b, x, 0, 0)),
        ],
        out_specs=pl.BlockSpec((1, XBLK, 64, RES),
                               lambda b, x: (b, x + 64 // XBLK, 1, 0)),
        out_shape=jax.ShapeDtypeStruct(
            (NUM_BATCHES, RES, RES, RES), jnp.float32),
        input_output_aliases={0: 0},
    )(base, sgrid)


def kernel(pts, bidx, occ_val, occ_val_grid):
    px = lax.slice_in_dim(pts, 0, 1, axis=1).reshape(N_PTS)
    py = lax.slice_in_dim(pts, 1, 2, axis=1).reshape(N_PTS)
    pz = lax.slice_in_dim(pts, 2, 3, axis=1).reshape(N_PTS)
    base = _copy(occ_val_grid)
    skey, sval, offs = _writer(px, py, pz, bidx, occ_val)
    neg = jnp.full((8, 64, 64), -1.0, jnp.float32)
    sgrid = _reader(neg, skey, sval, offs)
    return _octant(base, sgrid)


# speedup vs baseline: 1.1725x; 1.1725x over previous
import jax
import jax.numpy as jnp
from jax import lax
from jax.experimental import pallas as pl
from jax.experimental.pallas import tpu as pltpu
from jax.experimental.pallas import tpu_sc as plsc

N_PTS = 1048576
NUM_BATCHES = 8
RES = 128
EMA_DECAY = 0.95

NT = 32
NOWN = 64
CHUNK = N_PTS // NT
QTR = CHUNK // 4
NREG = N_PTS // QTR
WIN = 2048
C = 512
OROW = NOWN * 16 + 16


def _wid():
    return lax.axis_index("s") * 2 + lax.axis_index("c")


def _mesh():
    return plsc.VectorSubcoreMesh(core_axis_name="c", subcore_axis_name="s")


def _writer_body(px_hbm, py_hbm, pz_hbm, bidx_hbm, occ_hbm,
                 skey_hbm, sval_hbm, offs_hbm,
                 pxa, pya, pza, bxa, pxb, pyb, pzb, bxb,
                 keys, vals, hist, curs, offsb,
                 skey, sval, sem_a, sem_b, sem_v):
    wid = _wid()
    lane = lax.iota(jnp.int32, 16)
    c0 = jnp.zeros((16,), jnp.float32)
    ones = jnp.ones((16,), jnp.float32)
    NW = QTR // WIN

    def stage(q, w, px, py, pz, bx, sem):
        p0 = wid * CHUNK + q * QTR + w * WIN
        pltpu.async_copy(px_hbm.at[pl.ds(p0, WIN)], px, sem)
        pltpu.async_copy(py_hbm.at[pl.ds(p0, WIN)], py, sem)
        pltpu.async_copy(pz_hbm.at[pl.ds(p0, WIN)], pz, sem)
        pltpu.async_copy(bidx_hbm.at[pl.ds(p0, WIN)], bx, sem)
        pltpu.async_copy(occ_hbm.at[pl.ds(p0, WIN)],
                         vals.at[pl.ds(w * WIN, WIN)], sem_v)

    def wait4(px, py, pz, bx, sem):
        pltpu.make_async_copy(px_hbm.at[pl.ds(0, WIN)], px, sem).wait()
        pltpu.make_async_copy(px_hbm.at[pl.ds(0, WIN)], py, sem).wait()
        pltpu.make_async_copy(px_hbm.at[pl.ds(0, WIN)], pz, sem).wait()
        pltpu.make_async_copy(bidx_hbm.at[pl.ds(0, WIN)], bx, sem).wait()

    def quarter(q, _):
        def zh(j, _):
            hist[pl.ds(j * 16, 16)] = c0
            return 0
        lax.fori_loop(0, NOWN, zh, 0)

        def compute(w, px, py, pz, bx):
            def vreg(i, _):
                x = px[pl.ds(i * 16, 16)]
                y = py[pl.ds(i * 16, 16)]
                z = pz[pl.ds(i * 16, 16)]
                b = bx[pl.ds(i * 16, 16)]
                vx = jnp.clip((x * 64.0 + 64.0).astype(jnp.int32), 64, 127) - 64
                vy = jnp.clip((y * 64.0 + 64.0).astype(jnp.int32), 64, 127) - 64
                vz = jnp.clip((z * 64.0 + 64.0).astype(jnp.int32), 64, 127) - 64
                key = ((b * 64 + vx) * 64 + vy) * 64 + vz
                keys[pl.ds(w * WIN + i * 16, 16)] = key
                hidx = lax.shift_right_logical(key, 15) * 16 + lane
                plsc.addupdate_scatter(hist, [hidx], ones)
                return 0
            lax.fori_loop(0, WIN // 16, vreg, 0)

        stage(q, 0, pxa, pya, pza, bxa, sem_a)

        def winpair(wi, _):
            w = 2 * wi
            stage(q, w + 1, pxb, pyb, pzb, bxb, sem_b)
            wait4(pxa, pya, pza, bxa, sem_a)
            compute(w, pxa, pya, pza, bxa)

            @pl.when(w + 2 < NW)
            def _():
                stage(q, w + 2, pxa, pya, pza, bxa, sem_a)
            wait4(pxb, pyb, pzb, bxb, sem_b)
            compute(w + 1, pxb, pyb, pzb, bxb)
            return 0
        lax.fori_loop(0, NW // 2, winpair, 0)

        def drainv(w, _):
            pltpu.make_async_copy(occ_hbm.at[pl.ds(0, WIN)],
                                  vals.at[pl.ds(w * WIN, WIN)], sem_v).wait()
            return 0
        lax.fori_loop(0, NW, drainv, 0)

        def pfx(j, run):
            hv = hist[pl.ds(j * 16, 16)].astype(jnp.int32)
            inc = plsc.cumsum(hv)
            ex = inc - hv + run
            curs[pl.ds(j * 16, 16)] = ex
            offsb[pl.ds(j * 16, 16)] = ex
            return run + jnp.sum(hv)
        lax.fori_loop(0, NOWN, pfx, jnp.int32(0))

        def perm(i, _):
            k = keys[pl.ds(i * 16, 16)]
            v = vals[pl.ds(i * 16, 16)]
            hidx = lax.shift_right_logical(k, 15) * 16 + lane
            cur = plsc.load_gather(curs, [hidx])
            plsc.store_scatter(skey, [cur], k)
            plsc.store_scatter(sval, [cur], v)
            plsc.store_scatter(curs, [hidx], cur + 1)
            return 0
        lax.fori_loop(0, QTR // 16, perm, 0)

        r = wid * 4 + q
        pltpu.sync_copy(skey, skey_hbm.at[pl.ds(r * QTR, QTR)])
        pltpu.sync_copy(sval, sval_hbm.at[pl.ds(r * QTR, QTR)])
        pltpu.sync_copy(offsb, offs_hbm.at[pl.ds(r * OROW, OROW)])
        return 0

    lax.fori_loop(0, 4, quarter, 0)


def _writer(px, py, pz, bidx, occ_val):
    f = pl.kernel(
        _writer_body,
        out_type=(
            jax.ShapeDtypeStruct((N_PTS,), jnp.int32),
            jax.ShapeDtypeStruct((N_PTS,), jnp.float32),
            jax.ShapeDtypeStruct((NREG * OROW,), jnp.int32),
        ),
        mesh=_mesh(),
        compiler_params=pltpu.CompilerParams(needs_layout_passes=False),
        scratch_types=[
            pltpu.VMEM((WIN,), jnp.float32),
            pltpu.VMEM((WIN,), jnp.float32),
            pltpu.VMEM((WIN,), jnp.float32),
            pltpu.VMEM((WIN,), jnp.int32),
            pltpu.VMEM((WIN,), jnp.float32),
            pltpu.VMEM((WIN,), jnp.float32),
            pltpu.VMEM((WIN,), jnp.float32),
            pltpu.VMEM((WIN,), jnp.int32),
            pltpu.VMEM((QTR,), jnp.int32),
            pltpu.VMEM((QTR,), jnp.float32),
            pltpu.VMEM((NOWN * 16,), jnp.float32),
            pltpu.VMEM((NOWN * 16,), jnp.int32),
            pltpu.VMEM((OROW,), jnp.int32),
            pltpu.VMEM((QTR,), jnp.int32),
            pltpu.VMEM((QTR,), jnp.float32),
            pltpu.SemaphoreType.DMA,
            pltpu.SemaphoreType.DMA,
            pltpu.SemaphoreType.DMA,
        ],
    )
    return f(px, py, pz, bidx, occ_val)


GRP = 16


def _reader_body(neg_hbm, skey_hbm, sval_hbm, offs_hbm, sgrid_hbm,
                 s3, kbuf_a, vbuf_a, kbuf_b, vbuf_b,
                 kbuf_c, vbuf_c, kbuf_d, vbuf_d, offsg,
                 mystart, myend, sem_a, sem_b, sem_c, sem_d):
    wid = _wid()
    lane = lax.iota(jnp.int32, 16)
    zl = jnp.zeros((16,), jnp.int32)

    def owner(sub, _):
        o = wid * 2 + sub
        is_last = o == NOWN - 1

        pltpu.sync_copy(neg_hbm, s3)

        def group(g, _):
            pltpu.sync_copy(offs_hbm.at[pl.ds(g * GRP * OROW, GRP * OROW)],
                            offsg)
            gidx = lane * OROW + o * 16
            sv = plsc.load_gather(offsg, [gidx])
            ev = jnp.where(is_last, QTR, plsc.load_gather(offsg, [gidx + 16]))
            mystart[pl.ds(g * 16, 16)] = sv
            myend[pl.ds(g * 16, 16)] = ev
            return 0
        lax.fori_loop(0, NREG // GRP, group, 0)

        def bounds(r):
            ridx = zl + r
            start = plsc.load_gather(mystart, [ridx])[0]
            end = plsc.load_gather(myend, [ridx])[0]
            start8 = jnp.bitwise_and(start, ~7)
            nominal = r * QTR + start8
            goff = pl.multiple_of(jnp.minimum(nominal, N_PTS - C), 8)
            return start8, end, nominal, goff

        def issue(r, kb, vb, sem):
            _, _, _, goff = bounds(r)
            pltpu.async_copy(skey_hbm.at[pl.ds(goff, C)], kb, sem)
            pltpu.async_copy(sval_hbm.at[pl.ds(goff, C)], vb, sem)

        def do_vregs(kb, vb, va, vb_hi):
            def vpair(j, _):
                k1 = va + 2 * j
                k2 = jnp.minimum(k1 + 1, C // 16 - 1)
                kk1 = kb[pl.ds(k1 * 16, 16)]
                vv1 = vb[pl.ds(k1 * 16, 16)]
                kk2 = kb[pl.ds(k2 * 16, 16)]
                vv2 = vb[pl.ds(k2 * 16, 16)]
                m1 = jnp.logical_and(
                    lax.shift_right_logical(kk1, 15) == o, k1 < vb_hi)
                m2 = jnp.logical_and(
                    lax.shift_right_logical(kk2, 15) == o, k1 + 1 < vb_hi)
                rel1 = jnp.bitwise_and(kk1, 32767)
                a0 = lax.shift_right_logical(rel1, 12)
                a1 = jnp.bitwise_and(lax.shift_right_logical(rel1, 6), 63)
                a2 = jnp.bitwise_and(rel1, 63)
                rel2 = jnp.bitwise_and(kk2, 32767)
                b0 = lax.shift_right_logical(rel2, 12)
                b1 = jnp.bitwise_and(lax.shift_right_logical(rel2, 6), 63)
                b2 = jnp.bitwise_and(rel2, 63)

                def cond(c):
                    return jnp.logical_or(jnp.any(c[0]), jnp.any(c[1]))

                def body(c):
                    mm1, mm2 = c
                    cur1 = plsc.load_gather(s3, [a0, a1, a2], mask=mm1)
                    cur2 = plsc.load_gather(s3, [b0, b1, b2], mask=mm2)
                    up1 = jnp.logical_and(mm1, vv1 > cur1)
                    up2 = jnp.logical_and(mm2, vv2 > cur2)
                    plsc.store_scatter(s3, [a0, a1, a2], vv1, mask=up1)
                    plsc.store_scatter(s3, [b0, b1, b2], vv2, mask=up2)
                    re1 = plsc.load_gather(s3, [a0, a1, a2], mask=mm1)
                    re2 = plsc.load_gather(s3, [b0, b1, b2], mask=mm2)
                    return (jnp.logical_and(mm1, vv1 > re1),
                            jnp.logical_and(mm2, vv2 > re2))
                lax.while_loop(cond, body, (m1, m2))
                return 0
            npair = lax.shift_right_logical(vb_hi - va + 1, 1)
            lax.fori_loop(0, npair, vpair, 0)

        def process(r, kb, vb, sem):
            start8, end, nominal, goff = bounds(r)
            pltpu.make_async_copy(skey_hbm.at[pl.ds(0, C)], kb, sem).wait()
            pltpu.make_async_copy(sval_hbm.at[pl.ds(0, C)], vb, sem).wait()
            a = nominal - goff
            p1 = jnp.minimum(end, start8 + C)
            bb = r * QTR + p1 - goff
            do_vregs(kb, vb, lax.shift_right_logical(a, 4),
                     lax.shift_right_logical(bb + 15, 4))
            nch = (end - start8 + C - 1) // C

            def chunk(j, _):
                nom_j = r * QTR + start8 + j * C
                goff_j = pl.multiple_of(jnp.minimum(nom_j, N_PTS - C), 8)
                pltpu.sync_copy(skey_hbm.at[pl.ds(goff_j, C)], kb)
                pltpu.sync_copy(sval_hbm.at[pl.ds(goff_j, C)], vb)
                aj = nom_j - goff_j
                pj = jnp.minimum(end, start8 + (j + 1) * C)
                bj = r * QTR + pj - goff_j
                do_vregs(kb, vb, lax.shift_right_logical(aj, 4),
                         lax.shift_right_logical(bj + 15, 4))
                return 0
            lax.fori_loop(1, nch, chunk, 0)

        def rmap(r):
            return r

        issue(rmap(0), kbuf_a, vbuf_a, sem_a)
        issue(rmap(1), kbuf_b, vbuf_b, sem_b)
        issue(rmap(2), kbuf_c, vbuf_c, sem_c)

        def quad(t, _):
            r = 4 * t
            issue(rmap(r + 3), kbuf_d, vbuf_d, sem_d)
            process(rmap(r), kbuf_a, vbuf_a, sem_a)

            @pl.when(t < NREG // 4 - 1)
            def _():
                issue(rmap(r + 4), kbuf_a, vbuf_a, sem_a)
            process(rmap(r + 1), kbuf_b, vbuf_b, sem_b)

            @pl.when(t < NREG // 4 - 1)
            def _():
                issue(rmap(r + 5), kbuf_b, vbuf_b, sem_b)
            process(rmap(r + 2), kbuf_c, vbuf_c, sem_c)

            @pl.when(t < NREG // 4 - 1)
            def _():
                issue(rmap(r + 6), kbuf_c, vbuf_c, sem_c)
            process(rmap(r + 3), kbuf_d, vbuf_d, sem_d)
            return 0
        lax.fori_loop(0, NREG // 4, quad, 0)

        b = lax.shift_right_logical(o, 3)
        xi0 = jnp.bitwise_and(o, 7) * 8
        pltpu.sync_copy(s3, sgrid_hbm.at[b, pl.ds(xi0, 8)])
        return 0

    lax.fori_loop(0, 2, owner, 0)


def _reader(neg, skey, sval, offs):
    f = pl.kernel(
        _reader_body,
        out_type=jax.ShapeDtypeStruct((NUM_BATCHES, 64, 64, 64), jnp.float32),
        mesh=_mesh(),
        compiler_params=pltpu.CompilerParams(needs_layout_passes=False),
        scratch_types=[
            pltpu.VMEM((8, 64, 64), jnp.float32),
            pltpu.VMEM((C,), jnp.int32),
            pltpu.VMEM((C,), jnp.float32),
            pltpu.VMEM((C,), jnp.int32),
            pltpu.VMEM((C,), jnp.float32),
            pltpu.VMEM((C,), jnp.int32),
            pltpu.VMEM((C,), jnp.float32),
            pltpu.VMEM((C,), jnp.int32),
            pltpu.VMEM((C,), jnp.float32),
            pltpu.VMEM((GRP * OROW,), jnp.int32),
            pltpu.VMEM((NREG,), jnp.int32),
            pltpu.VMEM((NREG,), jnp.int32),
            pltpu.SemaphoreType.DMA,
            pltpu.SemaphoreType.DMA,
            pltpu.SemaphoreType.DMA,
            pltpu.SemaphoreType.DMA,
        ],
    )
    return f(neg, skey, sval, offs)


XBLK = 16


def _copy_body(g_ref, out_ref):
    out_ref[...] = g_ref[...]


def _copy(grid):
    return pl.pallas_call(
        _copy_body,
        grid=(NUM_BATCHES, RES // XBLK),
        in_specs=[pl.BlockSpec((1, XBLK, RES, RES), lambda b, x: (b, x, 0, 0))],
        out_specs=pl.BlockSpec((1, XBLK, RES, RES), lambda b, x: (b, x, 0, 0)),
        out_shape=jax.ShapeDtypeStruct(
            (NUM_BATCHES, RES, RES, RES), jnp.float32),
    )(grid)


def _oct_body(base_ref, s_ref, out_ref):
    gq = base_ref[0, :, :, 64:128]
    s = s_ref[0]
    out_ref[0, :, :, 0:64] = base_ref[0, :, :, 0:64]
    out_ref[0, :, :, 64:128] = jnp.where(
        s >= 0.0, jnp.maximum(jnp.float32(EMA_DECAY) * gq, s), gq)


def _octant(base, sgrid):
    return pl.pallas_call(
        _oct_body,
        grid=(NUM_BATCHES, 64 // XBLK),
        in_specs=[
            pl.BlockSpec((1, XBLK, 64, RES),
                         lambda b, x: (b, x + 64 // XBLK, 1, 0)),
            pl.BlockSpec((1, XBLK, 64, 64), lambda b, x: (b, x, 0, 0)),
        ],
        out_specs=pl.BlockSpec((1, XBLK, 64, RES),
                               lambda b, x: (b, x + 64 // XBLK, 1, 0)),
        out_shape=jax.ShapeDtypeStruct(
            (NUM_BATCHES, RES, RES, RES), jnp.float32),
        input_output_aliases={0: 0},
    )(base, sgrid)


def kernel(pts, bidx, occ_val, occ_val_grid):
    px = lax.slice_in_dim(pts, 0, 1, axis=1).reshape(N_PTS)
    py = lax.slice_in_dim(pts, 1, 2, axis=1).reshape(N_PTS)
    pz = lax.slice_in_dim(pts, 2, 3, axis=1).reshape(N_PTS)
    base = _copy(occ_val_grid)
    skey, sval, offs = _writer(px, py, pz, bidx, occ_val)
    neg = jnp.full((8, 64, 64), -1.0, jnp.float32)
    sgrid = _reader(neg, skey, sval, offs)
    return _octant(base, sgrid)


# speedup vs baseline: 1.2581x; 1.0730x over previous
import jax
import jax.numpy as jnp
from jax import lax
from jax.experimental import pallas as pl
from jax.experimental.pallas import tpu as pltpu
from jax.experimental.pallas import tpu_sc as plsc

N_PTS = 1048576
NUM_BATCHES = 8
RES = 128
EMA_DECAY = 0.95

NT = 32
NOWN = 64
CHUNK = N_PTS // NT
QTR = CHUNK // 4
NREG = N_PTS // QTR
WIN = 2048
C = 512
OROW = NOWN * 16 + 16


def _wid():
    return lax.axis_index("s") * 2 + lax.axis_index("c")


def _mesh():
    return plsc.VectorSubcoreMesh(core_axis_name="c", subcore_axis_name="s")


def _writer_body(px_hbm, py_hbm, pz_hbm, bidx_hbm, occ_hbm,
                 skey_hbm, sval_hbm, offs_hbm,
                 pxa, pya, pza, bxa, pxb, pyb, pzb, bxb,
                 keys, vals, hist, curs, offsb,
                 skey, sval, sem_a, sem_b, sem_v):
    wid = _wid()
    lane = lax.iota(jnp.int32, 16)
    c0 = jnp.zeros((16,), jnp.float32)
    ones = jnp.ones((16,), jnp.float32)
    NW = QTR // WIN

    def stage(q, w, px, py, pz, bx, sem):
        p0 = wid * CHUNK + q * QTR + w * WIN
        pltpu.async_copy(px_hbm.at[pl.ds(p0, WIN)], px, sem)
        pltpu.async_copy(py_hbm.at[pl.ds(p0, WIN)], py, sem)
        pltpu.async_copy(pz_hbm.at[pl.ds(p0, WIN)], pz, sem)
        pltpu.async_copy(bidx_hbm.at[pl.ds(p0, WIN)], bx, sem)
        pltpu.async_copy(occ_hbm.at[pl.ds(p0, WIN)],
                         vals.at[pl.ds(w * WIN, WIN)], sem_v)

    def wait4(px, py, pz, bx, sem):
        pltpu.make_async_copy(px_hbm.at[pl.ds(0, WIN)], px, sem).wait()
        pltpu.make_async_copy(px_hbm.at[pl.ds(0, WIN)], py, sem).wait()
        pltpu.make_async_copy(px_hbm.at[pl.ds(0, WIN)], pz, sem).wait()
        pltpu.make_async_copy(bidx_hbm.at[pl.ds(0, WIN)], bx, sem).wait()

    def quarter(q, _):
        def zh(j, _):
            hist[pl.ds(j * 16, 16)] = c0
            return 0
        lax.fori_loop(0, NOWN, zh, 0)

        def compute(w, px, py, pz, bx):
            def vreg(i, _):
                x = px[pl.ds(i * 16, 16)]
                y = py[pl.ds(i * 16, 16)]
                z = pz[pl.ds(i * 16, 16)]
                b = bx[pl.ds(i * 16, 16)]
                vx = jnp.clip((x * 64.0 + 64.0).astype(jnp.int32), 64, 127) - 64
                vy = jnp.clip((y * 64.0 + 64.0).astype(jnp.int32), 64, 127) - 64
                vz = jnp.clip((z * 64.0 + 64.0).astype(jnp.int32), 64, 127) - 64
                key = ((b * 64 + vx) * 64 + vy) * 64 + vz
                keys[pl.ds(w * WIN + i * 16, 16)] = key
                hidx = lax.shift_right_logical(key, 15) * 16 + lane
                plsc.addupdate_scatter(hist, [hidx], ones)
                return 0
            lax.fori_loop(0, WIN // 16, vreg, 0)

        stage(q, 0, pxa, pya, pza, bxa, sem_a)

        def winpair(wi, _):
            w = 2 * wi
            stage(q, w + 1, pxb, pyb, pzb, bxb, sem_b)
            wait4(pxa, pya, pza, bxa, sem_a)
            compute(w, pxa, pya, pza, bxa)

            @pl.when(w + 2 < NW)
            def _():
                stage(q, w + 2, pxa, pya, pza, bxa, sem_a)
            wait4(pxb, pyb, pzb, bxb, sem_b)
            compute(w + 1, pxb, pyb, pzb, bxb)
            return 0
        lax.fori_loop(0, NW // 2, winpair, 0)

        def drainv(w, _):
            pltpu.make_async_copy(occ_hbm.at[pl.ds(0, WIN)],
                                  vals.at[pl.ds(w * WIN, WIN)], sem_v).wait()
            return 0
        lax.fori_loop(0, NW, drainv, 0)

        def pfx(j, run):
            hv = hist[pl.ds(j * 16, 16)].astype(jnp.int32)
            inc = plsc.cumsum(hv)
            ex = inc - hv + run
            curs[pl.ds(j * 16, 16)] = ex
            offsb[pl.ds(j * 16, 16)] = ex
            return run + jnp.sum(hv)
        lax.fori_loop(0, NOWN, pfx, jnp.int32(0))

        def perm(i, _):
            k = keys[pl.ds(i * 16, 16)]
            v = vals[pl.ds(i * 16, 16)]
            hidx = lax.shift_right_logical(k, 15) * 16 + lane
            cur = plsc.load_gather(curs, [hidx])
            plsc.store_scatter(skey, [cur], k)
            plsc.store_scatter(sval, [cur], v)
            plsc.store_scatter(curs, [hidx], cur + 1)
            return 0
        lax.fori_loop(0, QTR // 16, perm, 0)

        r = wid * 4 + q
        pltpu.sync_copy(skey, skey_hbm.at[pl.ds(r * QTR, QTR)])
        pltpu.sync_copy(sval, sval_hbm.at[pl.ds(r * QTR, QTR)])
        pltpu.sync_copy(offsb, offs_hbm.at[pl.ds(r * OROW, OROW)])
        return 0

    lax.fori_loop(0, 4, quarter, 0)


def _writer(px, py, pz, bidx, occ_val):
    f = pl.kernel(
        _writer_body,
        out_type=(
            jax.ShapeDtypeStruct((N_PTS,), jnp.int32),
            jax.ShapeDtypeStruct((N_PTS,), jnp.float32),
            jax.ShapeDtypeStruct((NREG * OROW,), jnp.int32),
        ),
        mesh=_mesh(),
        compiler_params=pltpu.CompilerParams(needs_layout_passes=False),
        scratch_types=[
            pltpu.VMEM((WIN,), jnp.float32),
            pltpu.VMEM((WIN,), jnp.float32),
            pltpu.VMEM((WIN,), jnp.float32),
            pltpu.VMEM((WIN,), jnp.int32),
            pltpu.VMEM((WIN,), jnp.float32),
            pltpu.VMEM((WIN,), jnp.float32),
            pltpu.VMEM((WIN,), jnp.float32),
            pltpu.VMEM((WIN,), jnp.int32),
            pltpu.VMEM((QTR,), jnp.int32),
            pltpu.VMEM((QTR,), jnp.float32),
            pltpu.VMEM((NOWN * 16,), jnp.float32),
            pltpu.VMEM((NOWN * 16,), jnp.int32),
            pltpu.VMEM((OROW,), jnp.int32),
            pltpu.VMEM((QTR,), jnp.int32),
            pltpu.VMEM((QTR,), jnp.float32),
            pltpu.SemaphoreType.DMA,
            pltpu.SemaphoreType.DMA,
            pltpu.SemaphoreType.DMA,
        ],
    )
    return f(px, py, pz, bidx, occ_val)


GRP = 16


def _reader_body(neg_hbm, skey_hbm, sval_hbm, offs_hbm, sgrid_hbm,
                 s3, kbuf_a, vbuf_a, kbuf_b, vbuf_b,
                 kbuf_c, vbuf_c, kbuf_d, vbuf_d, offsg,
                 mystart, myend, sem_a, sem_b, sem_c, sem_d):
    wid = _wid()
    lane = lax.iota(jnp.int32, 16)
    zl = jnp.zeros((16,), jnp.int32)

    def owner(sub, _):
        o = wid * 2 + sub
        is_last = o == NOWN - 1

        pltpu.sync_copy(neg_hbm, s3)

        def group(g, _):
            pltpu.sync_copy(offs_hbm.at[pl.ds(g * GRP * OROW, GRP * OROW)],
                            offsg)
            gidx = lane * OROW + o * 16
            sv = plsc.load_gather(offsg, [gidx])
            ev = jnp.where(is_last, QTR, plsc.load_gather(offsg, [gidx + 16]))
            mystart[pl.ds(g * 16, 16)] = sv
            myend[pl.ds(g * 16, 16)] = ev
            return 0
        lax.fori_loop(0, NREG // GRP, group, 0)

        def bounds(r):
            ridx = zl + r
            start = plsc.load_gather(mystart, [ridx])[0]
            end = plsc.load_gather(myend, [ridx])[0]
            start8 = jnp.bitwise_and(start, ~7)
            nominal = r * QTR + start8
            goff = pl.multiple_of(jnp.minimum(nominal, N_PTS - C), 8)
            return start8, end, nominal, goff

        def issue(r, kb, vb, sem):
            _, _, _, goff = bounds(r)
            pltpu.async_copy(skey_hbm.at[pl.ds(goff, C)], kb, sem)
            pltpu.async_copy(sval_hbm.at[pl.ds(goff, C)], vb, sem)

        def do_vregs(kb, vb, va, vb_hi):
            def vquad(j, _):
                k1 = va + 4 * j
                ks = [k1, jnp.minimum(k1 + 1, C // 16 - 1),
                      jnp.minimum(k1 + 2, C // 16 - 1),
                      jnp.minimum(k1 + 3, C // 16 - 1)]
                kks = [kb[pl.ds(k * 16, 16)] for k in ks]
                vvs = [vb[pl.ds(k * 16, 16)] for k in ks]
                ms = tuple(
                    jnp.logical_and(
                        lax.shift_right_logical(kk, 15) == o, k1 + i < vb_hi)
                    for i, kk in enumerate(kks))
                idxs = []
                for kk in kks:
                    rel = jnp.bitwise_and(kk, 32767)
                    idxs.append([
                        lax.shift_right_logical(rel, 12),
                        jnp.bitwise_and(lax.shift_right_logical(rel, 6), 63),
                        jnp.bitwise_and(rel, 63)])

                def cond(c):
                    return jnp.any(c[0]) | jnp.any(c[1]) | jnp.any(c[2]) \
                        | jnp.any(c[3])

                def body(c):
                    curs_ = [plsc.load_gather(s3, idxs[i], mask=c[i])
                             for i in range(4)]
                    ups = [jnp.logical_and(c[i], vvs[i] > curs_[i])
                           for i in range(4)]
                    for i in range(4):
                        plsc.store_scatter(s3, idxs[i], vvs[i], mask=ups[i])
                    res = [plsc.load_gather(s3, idxs[i], mask=c[i])
                           for i in range(4)]
                    return tuple(jnp.logical_and(c[i], vvs[i] > res[i])
                                 for i in range(4))
                lax.while_loop(cond, body, ms)
                return 0
            nquad = lax.shift_right_logical(vb_hi - va + 3, 2)
            lax.fori_loop(0, nquad, vquad, 0)

        def process(r, kb, vb, sem):
            start8, end, nominal, goff = bounds(r)
            pltpu.make_async_copy(skey_hbm.at[pl.ds(0, C)], kb, sem).wait()
            pltpu.make_async_copy(sval_hbm.at[pl.ds(0, C)], vb, sem).wait()
            a = nominal - goff
            p1 = jnp.minimum(end, start8 + C)
            bb = r * QTR + p1 - goff
            do_vregs(kb, vb, lax.shift_right_logical(a, 4),
                     lax.shift_right_logical(bb + 15, 4))
            nch = (end - start8 + C - 1) // C

            def chunk(j, _):
                nom_j = r * QTR + start8 + j * C
                goff_j = pl.multiple_of(jnp.minimum(nom_j, N_PTS - C), 8)
                pltpu.sync_copy(skey_hbm.at[pl.ds(goff_j, C)], kb)
                pltpu.sync_copy(sval_hbm.at[pl.ds(goff_j, C)], vb)
                aj = nom_j - goff_j
                pj = jnp.minimum(end, start8 + (j + 1) * C)
                bj = r * QTR + pj - goff_j
                do_vregs(kb, vb, lax.shift_right_logical(aj, 4),
                         lax.shift_right_logical(bj + 15, 4))
                return 0
            lax.fori_loop(1, nch, chunk, 0)

        def rmap(r):
            return r

        issue(rmap(0), kbuf_a, vbuf_a, sem_a)
        issue(rmap(1), kbuf_b, vbuf_b, sem_b)
        issue(rmap(2), kbuf_c, vbuf_c, sem_c)

        def quad(t, _):
            r = 4 * t
            issue(rmap(r + 3), kbuf_d, vbuf_d, sem_d)
            process(rmap(r), kbuf_a, vbuf_a, sem_a)

            @pl.when(t < NREG // 4 - 1)
            def _():
                issue(rmap(r + 4), kbuf_a, vbuf_a, sem_a)
            process(rmap(r + 1), kbuf_b, vbuf_b, sem_b)

            @pl.when(t < NREG // 4 - 1)
            def _():
                issue(rmap(r + 5), kbuf_b, vbuf_b, sem_b)
            process(rmap(r + 2), kbuf_c, vbuf_c, sem_c)

            @pl.when(t < NREG // 4 - 1)
            def _():
                issue(rmap(r + 6), kbuf_c, vbuf_c, sem_c)
            process(rmap(r + 3), kbuf_d, vbuf_d, sem_d)
            return 0
        lax.fori_loop(0, NREG // 4, quad, 0)

        b = lax.shift_right_logical(o, 3)
        xi0 = jnp.bitwise_and(o, 7) * 8
        pltpu.sync_copy(s3, sgrid_hbm.at[b, pl.ds(xi0, 8)])
        return 0

    lax.fori_loop(0, 2, owner, 0)


def _reader(neg, skey, sval, offs):
    f = pl.kernel(
        _reader_body,
        out_type=jax.ShapeDtypeStruct((NUM_BATCHES, 64, 64, 64), jnp.float32),
        mesh=_mesh(),
        compiler_params=pltpu.CompilerParams(needs_layout_passes=False),
        scratch_types=[
            pltpu.VMEM((8, 64, 64), jnp.float32),
            pltpu.VMEM((C,), jnp.int32),
            pltpu.VMEM((C,), jnp.float32),
            pltpu.VMEM((C,), jnp.int32),
            pltpu.VMEM((C,), jnp.float32),
            pltpu.VMEM((C,), jnp.int32),
            pltpu.VMEM((C,), jnp.float32),
            pltpu.VMEM((C,), jnp.int32),
            pltpu.VMEM((C,), jnp.float32),
            pltpu.VMEM((GRP * OROW,), jnp.int32),
            pltpu.VMEM((NREG,), jnp.int32),
            pltpu.VMEM((NREG,), jnp.int32),
            pltpu.SemaphoreType.DMA,
            pltpu.SemaphoreType.DMA,
            pltpu.SemaphoreType.DMA,
            pltpu.SemaphoreType.DMA,
        ],
    )
    return f(neg, skey, sval, offs)


XBLK = 16


def _copy_body(g_ref, out_ref):
    out_ref[...] = g_ref[...]


def _copy(grid):
    return pl.pallas_call(
        _copy_body,
        grid=(NUM_BATCHES, RES // XBLK),
        in_specs=[pl.BlockSpec((1, XBLK, RES, RES), lambda b, x: (b, x, 0, 0))],
        out_specs=pl.BlockSpec((1, XBLK, RES, RES), lambda b, x: (b, x, 0, 0)),
        out_shape=jax.ShapeDtypeStruct(
            (NUM_BATCHES, RES, RES, RES), jnp.float32),
    )(grid)


def _oct_body(base_ref, s_ref, out_ref):
    gq = base_ref[0, :, :, 64:128]
    s = s_ref[0]
    out_ref[0, :, :, 0:64] = base_ref[0, :, :, 0:64]
    out_ref[0, :, :, 64:128] = jnp.where(
        s >= 0.0, jnp.maximum(jnp.float32(EMA_DECAY) * gq, s), gq)


def _octant(base, sgrid):
    return pl.pallas_call(
        _oct_body,
        grid=(NUM_BATCHES, 64 // XBLK),
        in_specs=[
            pl.BlockSpec((1, XBLK, 64, RES),
                         lambda b, x: (b, x + 64 // XBLK, 1, 0)),
            pl.BlockSpec((1, XBLK, 64, 64), lambda b, x: (b, x, 0, 0)),
        ],
        out_specs=pl.BlockSpec((1, XBLK, 64, RES),
                               lambda b, x: (b, x + 64 // XBLK, 1, 0)),
        out_shape=jax.ShapeDtypeStruct(
            (NUM_BATCHES, RES, RES, RES), jnp.float32),
        input_output_aliases={0: 0},
    )(base, sgrid)


def kernel(pts, bidx, occ_val, occ_val_grid):
    px = lax.slice_in_dim(pts, 0, 1, axis=1).reshape(N_PTS)
    py = lax.slice_in_dim(pts, 1, 2, axis=1).reshape(N_PTS)
    pz = lax.slice_in_dim(pts, 2, 3, axis=1).reshape(N_PTS)
    base = _copy(occ_val_grid)
    skey, sval, offs = _writer(px, py, pz, bidx, occ_val)
    neg = jnp.full((8, 64, 64), -1.0, jnp.float32)
    sgrid = _reader(neg, skey, sval, offs)
    return _octant(base, sgrid)


# speedup vs baseline: 1.3074x; 1.0392x over previous
import jax
import jax.numpy as jnp
from jax import lax
from jax.experimental import pallas as pl
from jax.experimental.pallas import tpu as pltpu
from jax.experimental.pallas import tpu_sc as plsc

N_PTS = 1048576
NUM_BATCHES = 8
RES = 128
EMA_DECAY = 0.95

NT = 32
NOWN = 64
CHUNK = N_PTS // NT
QTR = CHUNK // 4
NREG = N_PTS // QTR
WIN = 2048
C = 512
OROW = NOWN * 16 + 16


def _wid():
    return lax.axis_index("s") * 2 + lax.axis_index("c")


def _mesh():
    return plsc.VectorSubcoreMesh(core_axis_name="c", subcore_axis_name="s")


def _writer_body(px_hbm, py_hbm, pz_hbm, bidx_hbm, occ_hbm,
                 skey_hbm, sval_hbm, offs_hbm,
                 pxa, pya, pza, bxa, pxb, pyb, pzb, bxb,
                 keys, vals, hist, curs, offsb,
                 skey, sval, sem_a, sem_b, sem_v):
    wid = _wid()
    lane = lax.iota(jnp.int32, 16)
    c0 = jnp.zeros((16,), jnp.float32)
    ones = jnp.ones((16,), jnp.float32)
    NW = QTR // WIN

    def stage(q, w, px, py, pz, bx, sem):
        p0 = wid * CHUNK + q * QTR + w * WIN
        pltpu.async_copy(px_hbm.at[pl.ds(p0, WIN)], px, sem)
        pltpu.async_copy(py_hbm.at[pl.ds(p0, WIN)], py, sem)
        pltpu.async_copy(pz_hbm.at[pl.ds(p0, WIN)], pz, sem)
        pltpu.async_copy(bidx_hbm.at[pl.ds(p0, WIN)], bx, sem)
        pltpu.async_copy(occ_hbm.at[pl.ds(p0, WIN)],
                         vals.at[pl.ds(w * WIN, WIN)], sem_v)

    def wait4(px, py, pz, bx, sem):
        pltpu.make_async_copy(px_hbm.at[pl.ds(0, WIN)], px, sem).wait()
        pltpu.make_async_copy(px_hbm.at[pl.ds(0, WIN)], py, sem).wait()
        pltpu.make_async_copy(px_hbm.at[pl.ds(0, WIN)], pz, sem).wait()
        pltpu.make_async_copy(bidx_hbm.at[pl.ds(0, WIN)], bx, sem).wait()

    def quarter(q, _):
        def zh(j, _):
            hist[pl.ds(j * 16, 16)] = c0
            return 0
        lax.fori_loop(0, NOWN, zh, 0)

        def compute(w, px, py, pz, bx):
            def vreg(i, _):
                x = px[pl.ds(i * 16, 16)]
                y = py[pl.ds(i * 16, 16)]
                z = pz[pl.ds(i * 16, 16)]
                b = bx[pl.ds(i * 16, 16)]
                vx = jnp.clip((x * 64.0 + 64.0).astype(jnp.int32), 64, 127) - 64
                vy = jnp.clip((y * 64.0 + 64.0).astype(jnp.int32), 64, 127) - 64
                vz = jnp.clip((z * 64.0 + 64.0).astype(jnp.int32), 64, 127) - 64
                key = ((b * 64 + vx) * 64 + vy) * 64 + vz
                keys[pl.ds(w * WIN + i * 16, 16)] = key
                hidx = lax.shift_right_logical(key, 15) * 16 + lane
                plsc.addupdate_scatter(hist, [hidx], ones)
                return 0
            lax.fori_loop(0, WIN // 16, vreg, 0)

        stage(q, 0, pxa, pya, pza, bxa, sem_a)

        def winpair(wi, _):
            w = 2 * wi
            stage(q, w + 1, pxb, pyb, pzb, bxb, sem_b)
            wait4(pxa, pya, pza, bxa, sem_a)
            compute(w, pxa, pya, pza, bxa)

            @pl.when(w + 2 < NW)
            def _():
                stage(q, w + 2, pxa, pya, pza, bxa, sem_a)
            wait4(pxb, pyb, pzb, bxb, sem_b)
            compute(w + 1, pxb, pyb, pzb, bxb)
            return 0
        lax.fori_loop(0, NW // 2, winpair, 0)

        def drainv(w, _):
            pltpu.make_async_copy(occ_hbm.at[pl.ds(0, WIN)],
                                  vals.at[pl.ds(w * WIN, WIN)], sem_v).wait()
            return 0
        lax.fori_loop(0, NW, drainv, 0)

        def pfx(j, run):
            hv = hist[pl.ds(j * 16, 16)].astype(jnp.int32)
            inc = plsc.cumsum(hv)
            ex = inc - hv + run
            curs[pl.ds(j * 16, 16)] = ex
            offsb[pl.ds(j * 16, 16)] = ex
            return run + jnp.sum(hv)
        lax.fori_loop(0, NOWN, pfx, jnp.int32(0))

        def perm(i, _):
            k1 = keys[pl.ds(i * 32, 16)]
            v1 = vals[pl.ds(i * 32, 16)]
            k2 = keys[pl.ds(i * 32 + 16, 16)]
            v2 = vals[pl.ds(i * 32 + 16, 16)]
            h1 = lax.shift_right_logical(k1, 15) * 16 + lane
            h2 = lax.shift_right_logical(k2, 15) * 16 + lane
            c1 = plsc.load_gather(curs, [h1])
            c2r = plsc.load_gather(curs, [h2])
            c2 = jnp.where(h1 == h2, c1 + 1, c2r)
            plsc.store_scatter(skey, [c1], k1)
            plsc.store_scatter(sval, [c1], v1)
            plsc.store_scatter(skey, [c2], k2)
            plsc.store_scatter(sval, [c2], v2)
            plsc.store_scatter(curs, [h1], c1 + 1)
            plsc.store_scatter(curs, [h2], c2 + 1)
            return 0
        lax.fori_loop(0, QTR // 32, perm, 0)

        r = wid * 4 + q
        pltpu.sync_copy(skey, skey_hbm.at[pl.ds(r * QTR, QTR)])
        pltpu.sync_copy(sval, sval_hbm.at[pl.ds(r * QTR, QTR)])
        pltpu.sync_copy(offsb, offs_hbm.at[pl.ds(r * OROW, OROW)])
        return 0

    lax.fori_loop(0, 4, quarter, 0)


def _writer(px, py, pz, bidx, occ_val):
    f = pl.kernel(
        _writer_body,
        out_type=(
            jax.ShapeDtypeStruct((N_PTS,), jnp.int32),
            jax.ShapeDtypeStruct((N_PTS,), jnp.float32),
            jax.ShapeDtypeStruct((NREG * OROW,), jnp.int32),
        ),
        mesh=_mesh(),
        compiler_params=pltpu.CompilerParams(needs_layout_passes=False),
        scratch_types=[
            pltpu.VMEM((WIN,), jnp.float32),
            pltpu.VMEM((WIN,), jnp.float32),
            pltpu.VMEM((WIN,), jnp.float32),
            pltpu.VMEM((WIN,), jnp.int32),
            pltpu.VMEM((WIN,), jnp.float32),
            pltpu.VMEM((WIN,), jnp.float32),
            pltpu.VMEM((WIN,), jnp.float32),
            pltpu.VMEM((WIN,), jnp.int32),
            pltpu.VMEM((QTR,), jnp.int32),
            pltpu.VMEM((QTR,), jnp.float32),
            pltpu.VMEM((NOWN * 16,), jnp.float32),
            pltpu.VMEM((NOWN * 16,), jnp.int32),
            pltpu.VMEM((OROW,), jnp.int32),
            pltpu.VMEM((QTR,), jnp.int32),
            pltpu.VMEM((QTR,), jnp.float32),
            pltpu.SemaphoreType.DMA,
            pltpu.SemaphoreType.DMA,
            pltpu.SemaphoreType.DMA,
        ],
    )
    return f(px, py, pz, bidx, occ_val)


GRP = 16


def _reader_body(neg_hbm, skey_hbm, sval_hbm, offs_hbm, sgrid_hbm,
                 s3, kbuf_a, vbuf_a, kbuf_b, vbuf_b,
                 kbuf_c, vbuf_c, kbuf_d, vbuf_d, offsg,
                 mystart, myend, sem_a, sem_b, sem_c, sem_d):
    wid = _wid()
    lane = lax.iota(jnp.int32, 16)
    zl = jnp.zeros((16,), jnp.int32)

    def owner(sub, _):
        o = wid * 2 + sub
        is_last = o == NOWN - 1

        pltpu.sync_copy(neg_hbm, s3)

        def group(g, _):
            pltpu.sync_copy(offs_hbm.at[pl.ds(g * GRP * OROW, GRP * OROW)],
                            offsg)
            gidx = lane * OROW + o * 16
            sv = plsc.load_gather(offsg, [gidx])
            ev = jnp.where(is_last, QTR, plsc.load_gather(offsg, [gidx + 16]))
            mystart[pl.ds(g * 16, 16)] = sv
            myend[pl.ds(g * 16, 16)] = ev
            return 0
        lax.fori_loop(0, NREG // GRP, group, 0)

        def bounds(r):
            ridx = zl + r
            start = plsc.load_gather(mystart, [ridx])[0]
            end = plsc.load_gather(myend, [ridx])[0]
            start8 = jnp.bitwise_and(start, ~7)
            nominal = r * QTR + start8
            goff = pl.multiple_of(jnp.minimum(nominal, N_PTS - C), 8)
            return start8, end, nominal, goff

        def issue(r, kb, vb, sem):
            _, _, _, goff = bounds(r)
            pltpu.async_copy(skey_hbm.at[pl.ds(goff, C)], kb, sem)
            pltpu.async_copy(sval_hbm.at[pl.ds(goff, C)], vb, sem)

        def do_vregs(kb, vb, va, vb_hi):
            def vquad(j, _):
                k1 = va + 4 * j
                ks = [k1, jnp.minimum(k1 + 1, C // 16 - 1),
                      jnp.minimum(k1 + 2, C // 16 - 1),
                      jnp.minimum(k1 + 3, C // 16 - 1)]
                kks = [kb[pl.ds(k * 16, 16)] for k in ks]
                vvs = [vb[pl.ds(k * 16, 16)] for k in ks]
                ms = tuple(
                    jnp.logical_and(
                        lax.shift_right_logical(kk, 15) == o, k1 + i < vb_hi)
                    for i, kk in enumerate(kks))
                idxs = []
                for kk in kks:
                    rel = jnp.bitwise_and(kk, 32767)
                    idxs.append([
                        lax.shift_right_logical(rel, 12),
                        jnp.bitwise_and(lax.shift_right_logical(rel, 6), 63),
                        jnp.bitwise_and(rel, 63)])

                def cond(c):
                    return jnp.any(c[0]) | jnp.any(c[1]) | jnp.any(c[2]) \
                        | jnp.any(c[3])

                def body(c):
                    curs_ = [plsc.load_gather(s3, idxs[i], mask=c[i])
                             for i in range(4)]
                    ups = [jnp.logical_and(c[i], vvs[i] > curs_[i])
                           for i in range(4)]
                    for i in range(4):
                        plsc.store_scatter(s3, idxs[i], vvs[i], mask=ups[i])
                    res = [plsc.load_gather(s3, idxs[i], mask=c[i])
                           for i in range(4)]
                    return tuple(jnp.logical_and(c[i], vvs[i] > res[i])
                                 for i in range(4))
                lax.while_loop(cond, body, ms)
                return 0
            nquad = lax.shift_right_logical(vb_hi - va + 3, 2)
            lax.fori_loop(0, nquad, vquad, 0)

        def process(r, kb, vb, sem):
            start8, end, nominal, goff = bounds(r)
            pltpu.make_async_copy(skey_hbm.at[pl.ds(0, C)], kb, sem).wait()
            pltpu.make_async_copy(sval_hbm.at[pl.ds(0, C)], vb, sem).wait()
            a = nominal - goff
            p1 = jnp.minimum(end, start8 + C)
            bb = r * QTR + p1 - goff
            do_vregs(kb, vb, lax.shift_right_logical(a, 4),
                     lax.shift_right_logical(bb + 15, 4))
            nch = (end - start8 + C - 1) // C

            def chunk(j, _):
                nom_j = r * QTR + start8 + j * C
                goff_j = pl.multiple_of(jnp.minimum(nom_j, N_PTS - C), 8)
                pltpu.sync_copy(skey_hbm.at[pl.ds(goff_j, C)], kb)
                pltpu.sync_copy(sval_hbm.at[pl.ds(goff_j, C)], vb)
                aj = nom_j - goff_j
                pj = jnp.minimum(end, start8 + (j + 1) * C)
                bj = r * QTR + pj - goff_j
                do_vregs(kb, vb, lax.shift_right_logical(aj, 4),
                         lax.shift_right_logical(bj + 15, 4))
                return 0
            lax.fori_loop(1, nch, chunk, 0)

        def rmap(r):
            return r

        issue(rmap(0), kbuf_a, vbuf_a, sem_a)
        issue(rmap(1), kbuf_b, vbuf_b, sem_b)
        issue(rmap(2), kbuf_c, vbuf_c, sem_c)

        def quad(t, _):
            r = 4 * t
            issue(rmap(r + 3), kbuf_d, vbuf_d, sem_d)
            process(rmap(r), kbuf_a, vbuf_a, sem_a)

            @pl.when(t < NREG // 4 - 1)
            def _():
                issue(rmap(r + 4), kbuf_a, vbuf_a, sem_a)
            process(rmap(r + 1), kbuf_b, vbuf_b, sem_b)

            @pl.when(t < NREG // 4 - 1)
            def _():
                issue(rmap(r + 5), kbuf_b, vbuf_b, sem_b)
            process(rmap(r + 2), kbuf_c, vbuf_c, sem_c)

            @pl.when(t < NREG // 4 - 1)
            def _():
                issue(rmap(r + 6), kbuf_c, vbuf_c, sem_c)
            process(rmap(r + 3), kbuf_d, vbuf_d, sem_d)
            return 0
        lax.fori_loop(0, NREG // 4, quad, 0)

        b = lax.shift_right_logical(o, 3)
        xi0 = jnp.bitwise_and(o, 7) * 8
        pltpu.sync_copy(s3, sgrid_hbm.at[b, pl.ds(xi0, 8)])
        return 0

    lax.fori_loop(0, 2, owner, 0)


def _reader(neg, skey, sval, offs):
    f = pl.kernel(
        _reader_body,
        out_type=jax.ShapeDtypeStruct((NUM_BATCHES, 64, 64, 64), jnp.float32),
        mesh=_mesh(),
        compiler_params=pltpu.CompilerParams(needs_layout_passes=False),
        scratch_types=[
            pltpu.VMEM((8, 64, 64), jnp.float32),
            pltpu.VMEM((C,), jnp.int32),
            pltpu.VMEM((C,), jnp.float32),
            pltpu.VMEM((C,), jnp.int32),
            pltpu.VMEM((C,), jnp.float32),
            pltpu.VMEM((C,), jnp.int32),
            pltpu.VMEM((C,), jnp.float32),
            pltpu.VMEM((C,), jnp.int32),
            pltpu.VMEM((C,), jnp.float32),
            pltpu.VMEM((GRP * OROW,), jnp.int32),
            pltpu.VMEM((NREG,), jnp.int32),
            pltpu.VMEM((NREG,), jnp.int32),
            pltpu.SemaphoreType.DMA,
            pltpu.SemaphoreType.DMA,
            pltpu.SemaphoreType.DMA,
            pltpu.SemaphoreType.DMA,
        ],
    )
    return f(neg, skey, sval, offs)


XBLK = 16


def _copy_body(g_ref, out_ref):
    out_ref[...] = g_ref[...]


def _copy(grid):
    return pl.pallas_call(
        _copy_body,
        grid=(NUM_BATCHES, RES // XBLK),
        in_specs=[pl.BlockSpec((1, XBLK, RES, RES), lambda b, x: (b, x, 0, 0))],
        out_specs=pl.BlockSpec((1, XBLK, RES, RES), lambda b, x: (b, x, 0, 0)),
        out_shape=jax.ShapeDtypeStruct(
            (NUM_BATCHES, RES, RES, RES), jnp.float32),
    )(grid)


def _oct_body(base_ref, s_ref, out_ref):
    gq = base_ref[0, :, :, 64:128]
    s = s_ref[0]
    out_ref[0, :, :, 0:64] = base_ref[0, :, :, 0:64]
    out_ref[0, :, :, 64:128] = jnp.where(
        s >= 0.0, jnp.maximum(jnp.float32(EMA_DECAY) * gq, s), gq)


def _octant(base, sgrid):
    return pl.pallas_call(
        _oct_body,
        grid=(NUM_BATCHES, 64 // XBLK),
        in_specs=[
            pl.BlockSpec((1, XBLK, 64, RES),
                         lambda b, x: (b, x + 64 // XBLK, 1, 0)),
            pl.BlockSpec((1, XBLK, 64, 64), lambda b, x: (b, x, 0, 0)),
        ],
        out_specs=pl.BlockSpec((1, XBLK, 64, RES),
                               lambda b, x: (b, x + 64 // XBLK, 1, 0)),
        out_shape=jax.ShapeDtypeStruct(
            (NUM_BATCHES, RES, RES, RES), jnp.float32),
        input_output_aliases={0: 0},
    )(base, sgrid)


def kernel(pts, bidx, occ_val, occ_val_grid):
    px = lax.slice_in_dim(pts, 0, 1, axis=1).reshape(N_PTS)
    py = lax.slice_in_dim(pts, 1, 2, axis=1).reshape(N_PTS)
    pz = lax.slice_in_dim(pts, 2, 3, axis=1).reshape(N_PTS)
    base = _copy(occ_val_grid)
    skey, sval, offs = _writer(px, py, pz, bidx, occ_val)
    neg = jnp.full((8, 64, 64), -1.0, jnp.float32)
    sgrid = _reader(neg, skey, sval, offs)
    return _octant(base, sgrid)
